# Initial kernel scaffold; baseline (speedup 1.0000x reference)
import jax
import jax.numpy as jnp
from jax.experimental import pallas as pl


def kernel(dst, ref):
    raise NotImplementedError("write your pallas kernel here")


# speedup vs baseline: 135.2717x; 135.2717x over previous
import functools

import jax
import jax.numpy as jnp
from jax import lax
from jax.experimental import pallas as pl
from jax.experimental.pallas import tpu as pltpu
from jax.experimental.pallas import tpu_sc as plsc

HCH = (0, 1, 2, 3, 4, 6)
MPOS = (0, 0, 0, 0, 1, 2, 0, 2, 4, 0, 3, 5)

NC = 2
NS = 16
L = 16
NW = NC * NS
HW = 512 * 512
PIX_PER_W = HW // NW
GROUPS = PIX_PER_W // L
UNROLL = 8
NU = 2 * len(HCH)
WIDTH = 255.0 / 256.0

_mesh = plsc.VectorSubcoreMesh(core_axis_name="c", subcore_axis_name="s")
_cparams = pltpu.CompilerParams(needs_layout_passes=False)


@functools.partial(
    pl.kernel,
    out_type=jax.ShapeDtypeStruct((NU, NW, 256), jnp.float32),
    scratch_types=[
        pltpu.VMEM((NU * L * 256,), jnp.float32),
        pltpu.VMEM((PIX_PER_W,), jnp.float32),
        pltpu.VMEM((256,), jnp.float32),
    ],
    mesh=_mesh,
    compiler_params=_cparams,
)
def _hist_sc(dstp, refp, parts, histv, pixv, partv):
    wid = lax.axis_index("s") * NC + lax.axis_index("c")
    base = wid * PIX_PER_W
    lane_off = lax.iota(jnp.int32, L) * 256
    ones = jnp.ones((L,), jnp.float32)

    def zero_body(i, _):
        for g in range(UNROLL):
            histv[pl.ds((i * UNROLL + g) * L, L)] = jnp.zeros((L,), jnp.float32)
        return 0

    lax.fori_loop(0, (NU * L * 256) // (L * UNROLL), zero_body, 0)

    for u in range(NU):
        src = dstp if u < len(HCH) else refp
        ch = HCH[u % len(HCH)]
        pltpu.sync_copy(src.at[ch, pl.ds(base, PIX_PER_W)], pixv)
        hbase = u * L * 256

        def hist_body(i, _):
            for g in range(UNROLL):
                v = pixv[pl.ds((i * UNROLL + g) * L, L)]
                q = (v * 255.0) / WIDTH
                q = jnp.minimum(jnp.maximum(q, 0.0), 255.0)
                idx = q.astype(jnp.int32) + lane_off + hbase
                plsc.addupdate_scatter(histv, [idx], ones)
            return 0

        lax.fori_loop(0, GROUPS // UNROLL, hist_body, 0)

        def red_body(j, _):
            acc = histv[pl.ds(hbase + j * L, L)]
            for l in range(1, L):
                acc = acc + histv[pl.ds(hbase + l * 256 + j * L, L)]
            partv[pl.ds(j * L, L)] = acc
            return 0

        lax.fori_loop(0, 256 // L, red_body, 0)
        pltpu.sync_copy(partv, parts.at[u, wid])


def _table_body(parts_ref, lut_ref):
    parts = parts_ref[...]
    h = jnp.sum(parts, axis=1)
    hd = h[: len(HCH)]
    hr = h[len(HCH):]
    tri = (lax.broadcasted_iota(jnp.int32, (256, 256), 0)
           <= lax.broadcasted_iota(jnp.int32, (256, 256), 1)
           ).astype(jnp.float32)
    cd = jnp.dot(hd, tri, preferred_element_type=jnp.float32)
    cr = jnp.dot(hr, tri, preferred_element_type=jnp.float32)
    g = (cd[:, :, None] - cr[:, None, :] >= 0.0).astype(jnp.float32)
    tab = jnp.sum(g, axis=2) - 1.0
    tab = jnp.minimum(jnp.maximum(tab, 0.0), 255.0) * (1.0 / 255.0)
    lut_ref[...] = jnp.concatenate([tab[m][None] for m in MPOS], axis=0)


def _table_tc(parts):
    return pl.pallas_call(
        _table_body,
        out_shape=jax.ShapeDtypeStruct((12, 256), jnp.float32),
    )(parts)


@functools.partial(
    pl.kernel,
    out_type=jax.ShapeDtypeStruct((12, HW), jnp.float32),
    scratch_types=[
        pltpu.VMEM((12 * 256,), jnp.float32),
        pltpu.VMEM((PIX_PER_W,), jnp.float32),
        pltpu.VMEM((PIX_PER_W,), jnp.float32),
    ],
    mesh=_mesh,
    compiler_params=_cparams,
)
def _gather_sc(dstp, lutp, outp, lutv, pixv, outv):
    wid = lax.axis_index("s") * NC + lax.axis_index("c")
    base = wid * PIX_PER_W
    pltpu.sync_copy(lutp, lutv)
    for ch in range(12):
        pltpu.sync_copy(dstp.at[ch, pl.ds(base, PIX_PER_W)], pixv)
        cbase = ch * 256

        def body(i, _):
            for g in range(UNROLL):
                s = (i * UNROLL + g) * L
                v = pixv[pl.ds(s, L)]
                t = jnp.minimum(jnp.maximum(v * 255.0, 0.0), 255.0)
                idx = t.astype(jnp.int32) + cbase
                outv[pl.ds(s, L)] = plsc.load_gather(lutv, [idx])
            return 0

        lax.fori_loop(0, GROUPS // UNROLL, body, 0)
        pltpu.sync_copy(outv, outp.at[ch, pl.ds(base, PIX_PER_W)])


def kernel(dst, ref):
    B, C, H, W = dst.shape
    d2 = dst.reshape(B * C, H * W)
    r2 = ref.reshape(B * C, H * W)
    parts = _hist_sc(d2, r2)
    lut = _table_tc(parts)
    out = _gather_sc(d2, lut.reshape(12 * 256))
    return out.reshape(B, C, H, W)


# speedup vs baseline: 156.3563x; 1.1559x over previous
import functools

import jax
import jax.numpy as jnp
from jax import lax
from jax.experimental import pallas as pl
from jax.experimental.pallas import tpu as pltpu
from jax.experimental.pallas import tpu_sc as plsc

HCH = (0, 1, 2, 3, 4, 6)
MPOS = (0, 0, 0, 0, 1, 2, 0, 2, 4, 0, 3, 5)

NC = 2
NS = 16
L = 16
NW = NC * NS
HW = 512 * 512
PIX_PER_W = HW // NW
GROUPS = PIX_PER_W // L
UNROLL = 8
NU = 2 * len(HCH)

_mesh = plsc.VectorSubcoreMesh(core_axis_name="c", subcore_axis_name="s")
_cparams = pltpu.CompilerParams(needs_layout_passes=False)


@functools.partial(
    pl.kernel,
    out_type=jax.ShapeDtypeStruct((NW, NU, 256), jnp.float32),
    scratch_types=[
        pltpu.VMEM((NU * L * 256,), jnp.float32),
        pltpu.VMEM((PIX_PER_W,), jnp.float32),
        pltpu.VMEM((PIX_PER_W,), jnp.float32),
        pltpu.VMEM((NU, 256), jnp.float32),
        pltpu.SemaphoreType.DMA,
        pltpu.SemaphoreType.DMA,
    ],
    mesh=_mesh,
    compiler_params=_cparams,
)
def _hist_sc(dstp, refp, parts, histv, pix0, pix1, partv, sem0, sem1):
    wid = lax.axis_index("s") * NC + lax.axis_index("c")
    base = wid * PIX_PER_W
    lane_off = lax.iota(jnp.int32, L) * 256
    ones = jnp.ones((L,), jnp.float32)
    pixbufs = (pix0, pix1)
    sems = (sem0, sem1)

    def zero_body(i, _):
        for g in range(UNROLL):
            histv[pl.ds((i * UNROLL + g) * L, L)] = jnp.zeros((L,), jnp.float32)
        return 0

    lax.fori_loop(0, (NU * L * 256) // (L * UNROLL), zero_body, 0)

    def src_slice(u):
        src = dstp if u < len(HCH) else refp
        return src.at[HCH[u % len(HCH)], pl.ds(base, PIX_PER_W)]

    cps = [None] * NU
    cps[0] = pltpu.async_copy(src_slice(0), pix0, sem0)
    for u in range(NU):
        if u + 1 < NU:
            cps[u + 1] = pltpu.async_copy(
                src_slice(u + 1), pixbufs[(u + 1) % 2], sems[(u + 1) % 2])
        cps[u].wait()
        pixv = pixbufs[u % 2]
        laneu = lane_off + u * L * 256

        def hist_body(i, _):
            for g in range(UNROLL):
                v = pixv[pl.ds((i * UNROLL + g) * L, L)]
                q = jnp.minimum(jnp.maximum(v * 256.0, 0.0), 255.0)
                idx = q.astype(jnp.int32) + laneu
                plsc.addupdate_scatter(histv, [idx], ones)
            return 0

        lax.fori_loop(0, GROUPS // UNROLL, hist_body, 0)

    for u in range(NU):
        hbase = u * L * 256

        def red_body(j, _):
            acc = histv[pl.ds(hbase + j * L, L)]
            for l in range(1, L):
                acc = acc + histv[pl.ds(hbase + l * 256 + j * L, L)]
            partv[u, pl.ds(j * L, L)] = acc
            return 0

        lax.fori_loop(0, 256 // L, red_body, 0)
    pltpu.sync_copy(partv, parts.at[wid])


def _table_body(parts_ref, lut_ref):
    parts = parts_ref[...]
    h = jnp.sum(parts, axis=0)
    hd = h[: len(HCH)]
    hr = h[len(HCH):]
    tri = (lax.broadcasted_iota(jnp.int32, (256, 256), 0)
           <= lax.broadcasted_iota(jnp.int32, (256, 256), 1)
           ).astype(jnp.float32)
    cd = jnp.dot(hd, tri, preferred_element_type=jnp.float32)
    cr = jnp.dot(hr, tri, preferred_element_type=jnp.float32)
    g = (cd[:, :, None] - cr[:, None, :] >= 0.0).astype(jnp.float32)
    tab = jnp.sum(g, axis=2) - 1.0
    tab = jnp.minimum(jnp.maximum(tab, 0.0), 255.0) * (1.0 / 255.0)
    lut_ref[...] = jnp.concatenate([tab[m][None] for m in MPOS], axis=0)


def _table_tc(parts):
    return pl.pallas_call(
        _table_body,
        out_shape=jax.ShapeDtypeStruct((12, 256), jnp.float32),
    )(parts)


@functools.partial(
    pl.kernel,
    out_type=jax.ShapeDtypeStruct((12, HW), jnp.float32),
    scratch_types=[
        pltpu.VMEM((12 * 256,), jnp.float32),
        pltpu.VMEM((PIX_PER_W,), jnp.float32),
        pltpu.VMEM((PIX_PER_W,), jnp.float32),
        pltpu.VMEM((PIX_PER_W,), jnp.float32),
        pltpu.VMEM((PIX_PER_W,), jnp.float32),
        pltpu.SemaphoreType.DMA,
        pltpu.SemaphoreType.DMA,
        pltpu.SemaphoreType.DMA,
        pltpu.SemaphoreType.DMA,
    ],
    mesh=_mesh,
    compiler_params=_cparams,
)
def _gather_sc(dstp, lutp, outp, lutv, pix0, pix1, out0, out1,
               semi0, semi1, semo0, semo1):
    wid = lax.axis_index("s") * NC + lax.axis_index("c")
    base = wid * PIX_PER_W
    pixbufs = (pix0, pix1)
    outbufs = (out0, out1)
    isems = (semi0, semi1)
    osems = (semo0, semo1)
    pltpu.sync_copy(lutp, lutv)

    cpi = [None] * 12
    cpo = [None] * 12
    cpi[0] = pltpu.async_copy(dstp.at[0, pl.ds(base, PIX_PER_W)], pix0, semi0)
    for ch in range(12):
        if ch + 1 < 12:
            cpi[ch + 1] = pltpu.async_copy(
                dstp.at[ch + 1, pl.ds(base, PIX_PER_W)],
                pixbufs[(ch + 1) % 2], isems[(ch + 1) % 2])
        cpi[ch].wait()
        if ch >= 2:
            cpo[ch - 2].wait()
        pixv = pixbufs[ch % 2]
        outv = outbufs[ch % 2]
        cbase = ch * 256

        def body(i, _):
            for g in range(UNROLL):
                s = (i * UNROLL + g) * L
                v = pixv[pl.ds(s, L)]
                t = jnp.minimum(jnp.maximum(v * 255.0, 0.0), 255.0)
                idx = t.astype(jnp.int32) + cbase
                outv[pl.ds(s, L)] = plsc.load_gather(lutv, [idx])
            return 0

        lax.fori_loop(0, GROUPS // UNROLL, body, 0)
        cpo[ch] = pltpu.async_copy(
            outv, outp.at[ch, pl.ds(base, PIX_PER_W)], osems[ch % 2])
    cpo[10].wait()
    cpo[11].wait()


def kernel(dst, ref):
    B, C, H, W = dst.shape
    d2 = dst.reshape(B * C, H * W)
    r2 = ref.reshape(B * C, H * W)
    parts = _hist_sc(d2, r2)
    lut = _table_tc(parts)
    out = _gather_sc(d2, lut.reshape(12 * 256))
    return out.reshape(B, C, H, W)


# speedup vs baseline: 254.8136x; 1.6297x over previous
import functools

import jax
import jax.numpy as jnp
from jax import lax
from jax.experimental import pallas as pl
from jax.experimental.pallas import tpu as pltpu
from jax.experimental.pallas import tpu_sc as plsc

HCH = (0, 1, 2, 3, 4, 6)
MPOS = (0, 0, 0, 0, 1, 2, 0, 2, 4, 0, 3, 5)

NC = 2
NS = 16
L = 16
NW = NC * NS
HW = 512 * 512
PIX_PER_W = HW // NW
GROUPS = PIX_PER_W // L
UNROLL = 8
NU = 2 * len(HCH)

_mesh = plsc.VectorSubcoreMesh(core_axis_name="c", subcore_axis_name="s")
_cparams = pltpu.CompilerParams(needs_layout_passes=False)


@functools.partial(
    pl.kernel,
    out_type=jax.ShapeDtypeStruct((NW, NU, 256), jnp.float32),
    scratch_types=[
        pltpu.VMEM((NU * L * 256,), jnp.float32),
        pltpu.VMEM((PIX_PER_W,), jnp.float32),
        pltpu.VMEM((PIX_PER_W,), jnp.float32),
        pltpu.VMEM((NU, 256), jnp.float32),
        pltpu.SemaphoreType.DMA,
        pltpu.SemaphoreType.DMA,
    ],
    mesh=_mesh,
    compiler_params=_cparams,
)
def _hist_sc(dstp, refp, parts, histv, pix0, pix1, partv, sem0, sem1):
    wid = lax.axis_index("s") * NC + lax.axis_index("c")
    base = wid * PIX_PER_W
    lane_off = lax.iota(jnp.int32, L) * 256
    ones = jnp.ones((L,), jnp.float32)
    pixbufs = (pix0, pix1)
    sems = (sem0, sem1)

    @plsc.parallel_loop(0, (NU * L * 256) // L, unroll=UNROLL)
    def zero_body(i):
        histv[pl.ds(i * L, L)] = jnp.zeros((L,), jnp.float32)

    def src_slice(u):
        src = dstp if u < len(HCH) else refp
        return src.at[HCH[u % len(HCH)], pl.ds(base, PIX_PER_W)]

    cps = [None] * NU
    cps[0] = pltpu.async_copy(src_slice(0), pix0, sem0)
    for u in range(NU):
        if u + 1 < NU:
            cps[u + 1] = pltpu.async_copy(
                src_slice(u + 1), pixbufs[(u + 1) % 2], sems[(u + 1) % 2])
        cps[u].wait()
        pixv = pixbufs[u % 2]
        laneu = lane_off + u * L * 256

        @plsc.parallel_loop(0, GROUPS, unroll=UNROLL)
        def hist_body(i):
            v = pixv[pl.ds(i * L, L)]
            q = jnp.minimum(jnp.maximum(v * 256.0, 0.0), 255.0)
            idx = q.astype(jnp.int32) + laneu
            plsc.addupdate_scatter(histv, [idx], ones)

    for u in range(NU):
        hbase = u * L * 256

        @plsc.parallel_loop(0, 256 // L, unroll=2)
        def red_body(j):
            acc = histv[pl.ds(hbase + j * L, L)]
            for l in range(1, L):
                acc = acc + histv[pl.ds(hbase + l * 256 + j * L, L)]
            partv[u, pl.ds(j * L, L)] = acc
    pltpu.sync_copy(partv, parts.at[wid])


def _table_body(parts_ref, lut_ref):
    parts = parts_ref[...]
    h = jnp.sum(parts, axis=0)
    hd = h[: len(HCH)]
    hr = h[len(HCH):]
    tri = (lax.broadcasted_iota(jnp.int32, (256, 256), 0)
           <= lax.broadcasted_iota(jnp.int32, (256, 256), 1)
           ).astype(jnp.float32)
    cd = jnp.dot(hd, tri, preferred_element_type=jnp.float32)
    cr = jnp.dot(hr, tri, preferred_element_type=jnp.float32)
    g = (cd[:, :, None] - cr[:, None, :] >= 0.0).astype(jnp.float32)
    tab = jnp.sum(g, axis=2) - 1.0
    tab = jnp.minimum(jnp.maximum(tab, 0.0), 255.0) * (1.0 / 255.0)
    lut_ref[...] = jnp.concatenate([tab[m][None] for m in MPOS], axis=0)


def _table_tc(parts):
    return pl.pallas_call(
        _table_body,
        out_shape=jax.ShapeDtypeStruct((12, 256), jnp.float32),
    )(parts)


@functools.partial(
    pl.kernel,
    out_type=jax.ShapeDtypeStruct((12, HW), jnp.float32),
    scratch_types=[
        pltpu.VMEM((12 * 256,), jnp.float32),
        pltpu.VMEM((PIX_PER_W,), jnp.float32),
        pltpu.VMEM((PIX_PER_W,), jnp.float32),
        pltpu.VMEM((PIX_PER_W,), jnp.float32),
        pltpu.VMEM((PIX_PER_W,), jnp.float32),
        pltpu.SemaphoreType.DMA,
        pltpu.SemaphoreType.DMA,
        pltpu.SemaphoreType.DMA,
        pltpu.SemaphoreType.DMA,
    ],
    mesh=_mesh,
    compiler_params=_cparams,
)
def _gather_sc(dstp, lutp, outp, lutv, pix0, pix1, out0, out1,
               semi0, semi1, semo0, semo1):
    wid = lax.axis_index("s") * NC + lax.axis_index("c")
    base = wid * PIX_PER_W
    pixbufs = (pix0, pix1)
    outbufs = (out0, out1)
    isems = (semi0, semi1)
    osems = (semo0, semo1)
    pltpu.sync_copy(lutp, lutv)

    cpi = [None] * 12
    cpo = [None] * 12
    cpi[0] = pltpu.async_copy(dstp.at[0, pl.ds(base, PIX_PER_W)], pix0, semi0)
    for ch in range(12):
        if ch + 1 < 12:
            cpi[ch + 1] = pltpu.async_copy(
                dstp.at[ch + 1, pl.ds(base, PIX_PER_W)],
                pixbufs[(ch + 1) % 2], isems[(ch + 1) % 2])
        cpi[ch].wait()
        if ch >= 2:
            cpo[ch - 2].wait()
        pixv = pixbufs[ch % 2]
        outv = outbufs[ch % 2]
        cbase = ch * 256

        @plsc.parallel_loop(0, GROUPS, unroll=UNROLL)
        def body(i):
            s = i * L
            v = pixv[pl.ds(s, L)]
            t = jnp.minimum(jnp.maximum(v * 255.0, 0.0), 255.0)
            idx = t.astype(jnp.int32) + cbase
            outv[pl.ds(s, L)] = plsc.load_gather(lutv, [idx])
        cpo[ch] = pltpu.async_copy(
            outv, outp.at[ch, pl.ds(base, PIX_PER_W)], osems[ch % 2])
    cpo[10].wait()
    cpo[11].wait()


def kernel(dst, ref):
    B, C, H, W = dst.shape
    d2 = dst.reshape(B * C, H * W)
    r2 = ref.reshape(B * C, H * W)
    parts = _hist_sc(d2, r2)
    lut = _table_tc(parts)
    out = _gather_sc(d2, lut.reshape(12 * 256))
    return out.reshape(B, C, H, W)


# speedup vs baseline: 517.0731x; 2.0292x over previous
import functools

import jax
import jax.numpy as jnp
from jax import lax
from jax.experimental import pallas as pl
from jax.experimental.pallas import tpu as pltpu
from jax.experimental.pallas import tpu_sc as plsc

HCH = (0, 1, 2, 3, 4, 6)
MPOS = (0, 0, 0, 0, 1, 2, 0, 2, 4, 0, 3, 5)

NC = 2
NS = 16
L = 16
NW = NC * NS
H = W = 512
ROWS_PER_W = H // NW
PIX_PER_W = ROWS_PER_W * W
GROUPS = PIX_PER_W // L
GPR = W // L
UNROLL = 8
NU = 2 * len(HCH)

_mesh = plsc.VectorSubcoreMesh(core_axis_name="c", subcore_axis_name="s")
_cparams = pltpu.CompilerParams(needs_layout_passes=False)


@functools.partial(
    pl.kernel,
    out_type=jax.ShapeDtypeStruct((NW, NU, 256), jnp.float32),
    scratch_types=[
        pltpu.VMEM((NU * L * 256,), jnp.float32),
        pltpu.VMEM((ROWS_PER_W, W), jnp.float32),
        pltpu.VMEM((ROWS_PER_W, W), jnp.float32),
        pltpu.VMEM((NU, 256), jnp.float32),
        pltpu.SemaphoreType.DMA,
        pltpu.SemaphoreType.DMA,
    ],
    mesh=_mesh,
    compiler_params=_cparams,
)
def _hist_sc(dstp, refp, parts, histv, pix0, pix1, partv, sem0, sem1):
    wid = lax.axis_index("s") * NC + lax.axis_index("c")
    rbase = wid * ROWS_PER_W
    lane_off = lax.iota(jnp.int32, L) * 256
    ones = jnp.ones((L,), jnp.float32)
    pixbufs = (pix0, pix1)
    sems = (sem0, sem1)

    @plsc.parallel_loop(0, (NU * L * 256) // L, unroll=UNROLL)
    def zero_body(i):
        histv[pl.ds(i * L, L)] = jnp.zeros((L,), jnp.float32)

    def src_slice(u):
        src = dstp if u < len(HCH) else refp
        b, c = divmod(HCH[u % len(HCH)], 3)
        return src.at[b, c, pl.ds(rbase, ROWS_PER_W), :]

    cps = [None] * NU
    cps[0] = pltpu.async_copy(src_slice(0), pix0, sem0)
    for u in range(NU):
        if u + 1 < NU:
            cps[u + 1] = pltpu.async_copy(
                src_slice(u + 1), pixbufs[(u + 1) % 2], sems[(u + 1) % 2])
        cps[u].wait()
        pixv = pixbufs[u % 2]
        laneu = lane_off + u * L * 256

        @plsc.parallel_loop(0, GROUPS, unroll=UNROLL)
        def hist_body(i):
            r = i // GPR
            col = (i % GPR) * L
            v = pixv[r, pl.ds(col, L)]
            q = jnp.minimum(jnp.maximum(v * 256.0, 0.0), 255.0)
            idx = q.astype(jnp.int32) + laneu
            plsc.addupdate_scatter(histv, [idx], ones)

    for u in range(NU):
        hbase = u * L * 256

        @plsc.parallel_loop(0, 256 // L, unroll=2)
        def red_body(j):
            acc = histv[pl.ds(hbase + j * L, L)]
            for l in range(1, L):
                acc = acc + histv[pl.ds(hbase + l * 256 + j * L, L)]
            partv[u, pl.ds(j * L, L)] = acc

    pltpu.sync_copy(partv, parts.at[wid])


def _table_body(parts_ref, lut_ref):
    parts = parts_ref[...]
    h = jnp.sum(parts, axis=0)
    hd = h[: len(HCH)]
    hr = h[len(HCH):]
    tri = (lax.broadcasted_iota(jnp.int32, (256, 256), 0)
           <= lax.broadcasted_iota(jnp.int32, (256, 256), 1)
           ).astype(jnp.float32)
    cd = jnp.dot(hd, tri, preferred_element_type=jnp.float32)
    cr = jnp.dot(hr, tri, preferred_element_type=jnp.float32)
    g = (cd[:, :, None] - cr[:, None, :] >= 0.0).astype(jnp.float32)
    tab = jnp.sum(g, axis=2) - 1.0
    tab = jnp.minimum(jnp.maximum(tab, 0.0), 255.0) * (1.0 / 255.0)
    lut_ref[...] = jnp.concatenate([tab[m][None] for m in MPOS], axis=0)


def _table_tc(parts):
    return pl.pallas_call(
        _table_body,
        out_shape=jax.ShapeDtypeStruct((12, 256), jnp.float32),
    )(parts)


@functools.partial(
    pl.kernel,
    out_type=jax.ShapeDtypeStruct((4, 3, H, W), jnp.float32),
    scratch_types=[
        pltpu.VMEM((12 * 256,), jnp.float32),
        pltpu.VMEM((ROWS_PER_W, W), jnp.float32),
        pltpu.VMEM((ROWS_PER_W, W), jnp.float32),
        pltpu.VMEM((ROWS_PER_W, W), jnp.float32),
        pltpu.VMEM((ROWS_PER_W, W), jnp.float32),
        pltpu.SemaphoreType.DMA,
        pltpu.SemaphoreType.DMA,
        pltpu.SemaphoreType.DMA,
        pltpu.SemaphoreType.DMA,
    ],
    mesh=_mesh,
    compiler_params=_cparams,
)
def _gather_sc(dstp, lutp, outp, lutv, pix0, pix1, out0, out1,
               semi0, semi1, semo0, semo1):
    wid = lax.axis_index("s") * NC + lax.axis_index("c")
    rbase = wid * ROWS_PER_W
    pixbufs = (pix0, pix1)
    outbufs = (out0, out1)
    isems = (semi0, semi1)
    osems = (semo0, semo1)
    pltpu.sync_copy(lutp, lutv)

    cpi = [None] * 12
    cpo = [None] * 12
    cpi[0] = pltpu.async_copy(
        dstp.at[0, 0, pl.ds(rbase, ROWS_PER_W), :], pix0, semi0)
    for ch in range(12):
        if ch + 1 < 12:
            b, c = divmod(ch + 1, 3)
            cpi[ch + 1] = pltpu.async_copy(
                dstp.at[b, c, pl.ds(rbase, ROWS_PER_W), :],
                pixbufs[(ch + 1) % 2], isems[(ch + 1) % 2])
        cpi[ch].wait()
        if ch >= 2:
            cpo[ch - 2].wait()
        pixv = pixbufs[ch % 2]
        outv = outbufs[ch % 2]
        cbase = ch * 256

        @plsc.parallel_loop(0, GROUPS, unroll=UNROLL)
        def body(i):
            r = i // GPR
            col = (i % GPR) * L
            v = pixv[r, pl.ds(col, L)]
            t = jnp.minimum(jnp.maximum(v * 255.0, 0.0), 255.0)
            idx = t.astype(jnp.int32) + cbase
            outv[r, pl.ds(col, L)] = plsc.load_gather(lutv, [idx])

        b, c = divmod(ch, 3)
        cpo[ch] = pltpu.async_copy(
            outv, outp.at[b, c, pl.ds(rbase, ROWS_PER_W), :], osems[ch % 2])
    cpo[10].wait()
    cpo[11].wait()


def kernel(dst, ref):
    parts = _hist_sc(dst, ref)
    lut = _table_tc(parts)
    return _gather_sc(dst, lut.reshape(12 * 256))
